# baseline (speedup 1.0000x reference)
import functools

import jax
import jax.numpy as jnp
from jax import lax
from jax.experimental import pallas as pl
from jax.experimental.pallas import tpu as pltpu
from jax.experimental.pallas import tpu_sc as plsc

N_NEWS = 10000
N_ENTITY = 30000
N_NODES = N_NEWS + N_ENTITY
N_USERS = 4096
N_FACTORS = 8
N_REL = 40
D = 100
Dp = 128
N_NEIGH = 20
NNZ = 262144

L = 16
NC = 2
NS = 16
NW = NC * NS
BN = 80
NBLK_NEWS = N_NEWS // BN
NBLK_ENT = N_ENTITY // BN
UPW = N_USERS // NW
CB = 128
NCH = Dp // L

_mesh = plsc.VectorSubcoreMesh(core_axis_name="c", subcore_axis_name="s")


def _worker_id():
    return lax.axis_index("s") * NC + lax.axis_index("c")


@functools.partial(
    pl.kernel,
    out_type=jax.ShapeDtypeStruct((N_NODES, Dp), jnp.float32),
    mesh=_mesh,
    scratch_types=[
        pltpu.VMEM((2 * N_NEIGH * BN,), jnp.int32),
        pltpu.VMEM((2 * N_NEIGH * BN,), jnp.int32),
        pltpu.VMEM((BN, Dp), jnp.float32),
        pltpu.VMEM((BN, Dp), jnp.float32),
        pltpu.SemaphoreType.DMA,
        pltpu.SemaphoreType.DMA,
    ],
)
def _node_kernel(ent_hbm, all_hbm, rel_hbm, nidx_hbm, ecidx_hbm, agg_hbm,
                 idx0, idx1, g0, g1, sem0, sem1):
    w = _worker_id()

    def _phase(nblk, idx_hbm, table_hbm, row_off, n_rel_streams):
        nb = (nblk - w + NW - 1) // NW
        slab = (N_NEIGH + n_rel_streams) * BN

        def issue(m, ib, gb, sem):
            @pl.loop(0, BN)
            def _z(r):
                for ci in range(NCH):
                    gb[r, pl.ds(ci * L, L)] = jnp.zeros((L,), jnp.float32)

            b = w + m * NW
            pltpu.sync_copy(idx_hbm.at[pl.ds(b * slab, slab)],
                            ib.at[pl.ds(0, slab)])
            for j in range(N_NEIGH):
                pltpu.async_copy(table_hbm.at[ib.at[pl.ds(j * BN, BN)]], gb,
                                 sem, add=True)
            for j in range(N_NEIGH, N_NEIGH + n_rel_streams):
                pltpu.async_copy(rel_hbm.at[ib.at[pl.ds(j * BN, BN)]], gb,
                                 sem, add=True)

        def finish(m, ib, gb, sem):
            base = (w + m * NW) * BN
            for j in range(N_NEIGH):
                pltpu.make_async_copy(
                    table_hbm.at[ib.at[pl.ds(j * BN, BN)]], gb, sem).wait()
            for j in range(N_NEIGH, N_NEIGH + n_rel_streams):
                pltpu.make_async_copy(
                    rel_hbm.at[ib.at[pl.ds(j * BN, BN)]], gb, sem).wait()
            pltpu.sync_copy(gb, agg_hbm.at[pl.ds(row_off + base, BN)])

        issue(0, idx0, g0, sem0)

        @pl.loop(0, (nb + 1) // 2)
        def _pair(p):
            m0 = 2 * p
            m1 = m0 + 1

            @pl.when(m1 < nb)
            def _():
                issue(m1, idx1, g1, sem1)

            finish(m0, idx0, g0, sem0)

            @pl.when(m0 + 2 < nb)
            def _():
                issue(m0 + 2, idx0, g0, sem0)

            @pl.when(m1 < nb)
            def _():
                finish(m1, idx1, g1, sem1)

    _phase(NBLK_NEWS, nidx_hbm, ent_hbm, 0, 0)
    _phase(NBLK_ENT, ecidx_hbm, all_hbm, N_NEWS, N_NEIGH)


NNZ_PER_TILE = NNZ // NW
NCHUNK = NNZ_PER_TILE // CB
URT = N_USERS // NS


@functools.partial(
    pl.kernel,
    out_type=jax.ShapeDtypeStruct((NC, N_USERS, Dp), jnp.float32),
    mesh=_mesh,
    scratch_types=[
        pltpu.VMEM((CB,), jnp.int32),
        pltpu.VMEM((CB,), jnp.int32),
        pltpu.VMEM((CB,), jnp.int32),
        pltpu.VMEM((CB,), jnp.int32),
        pltpu.VMEM((CB,), jnp.float32),
        pltpu.VMEM((CB,), jnp.float32),
        pltpu.VMEM((CB, Dp), jnp.float32),
        pltpu.VMEM((CB, Dp), jnp.float32),
        pltpu.VMEM_SHARED((N_USERS, Dp), jnp.float32),
        pltpu.SemaphoreType.DMA,
        pltpu.SemaphoreType.DMA,
        pltpu.SemaphoreType.DMA,
        pltpu.SemaphoreType.DMA,
    ],
)
def _user_kernel(node_hbm, cols_hbm, vals_hbm, rows_hbm, out_hbm,
                 idx0, idx1, row0, row1, val0, val1, g0, g1, acc_sh,
                 semg0, semg1, sems0, sems1):
    c = lax.axis_index("c")
    s = lax.axis_index("s")
    gid = s * NC + c
    base = gid * NNZ_PER_TILE

    @pl.loop(0, CB)
    def _z(r):
        for ci in range(NCH):
            g0[r, pl.ds(ci * L, L)] = jnp.zeros((L,), jnp.float32)

    pltpu.sync_copy(g0, acc_sh.at[pl.ds(s * URT, CB)])
    pltpu.sync_copy(g0, acc_sh.at[pl.ds(s * URT + CB, CB)])
    plsc.subcore_barrier()

    def issue(k, ib, rb, vb, gb, semg, sems, drain):
        if drain:
            pltpu.make_async_copy(node_hbm.at[pl.ds(0, CB)], gb, sems).wait()
        cbase = base + k * CB
        pltpu.sync_copy(cols_hbm.at[pl.ds(cbase, CB)], ib)
        pltpu.sync_copy(rows_hbm.at[pl.ds(cbase, CB)], rb)
        pltpu.sync_copy(vals_hbm.at[pl.ds(cbase, CB)], vb)
        pltpu.async_copy(node_hbm.at[ib], gb, semg)

    def finish(k, ib, rb, vb, gb, semg, sems):
        pltpu.make_async_copy(node_hbm.at[ib], gb, semg).wait()

        @pl.loop(0, CB // L)
        def _group(g):
            vvec = vb[pl.ds(g * L, L)]
            for j in range(L):
                r = g * L + j
                vv = vvec[j]
                for ci in range(NCH - 1):
                    sl = pl.ds(ci * L, L)
                    gb[r, sl] = gb[r, sl] * vv

        pltpu.async_copy(gb, acc_sh.at[rb], sems, add=True)

    issue(0, idx0, row0, val0, g0, semg0, sems0, drain=False)
    issue(1, idx1, row1, val1, g1, semg1, sems1, drain=False)

    @pl.loop(0, NCHUNK // 2)
    def _pair(p):
        k0 = 2 * p
        k1 = k0 + 1
        finish(k0, idx0, row0, val0, g0, semg0, sems0)

        @pl.when(k0 + 2 < NCHUNK)
        def _():
            issue(k0 + 2, idx0, row0, val0, g0, semg0, sems0, drain=True)

        finish(k1, idx1, row1, val1, g1, semg1, sems1)

        @pl.when(k1 + 2 < NCHUNK)
        def _():
            issue(k1 + 2, idx1, row1, val1, g1, semg1, sems1, drain=True)

    pltpu.make_async_copy(node_hbm.at[pl.ds(0, CB)], g0, sems0).wait()
    pltpu.make_async_copy(node_hbm.at[pl.ds(0, CB)], g1, sems1).wait()
    plsc.subcore_barrier()
    pltpu.sync_copy(acc_sh.at[pl.ds(s * URT, URT)],
                    out_hbm.at[c].at[pl.ds(s * URT, URT)])


@functools.partial(
    pl.kernel,
    out_type=jax.ShapeDtypeStruct((N_USERS, Dp), jnp.float32),
    mesh=_mesh,
    scratch_types=[
        pltpu.VMEM((UPW, Dp), jnp.float32),
        pltpu.VMEM((UPW, Dp), jnp.float32),
        pltpu.VMEM((UPW, Dp), jnp.float32),
    ],
)
def _combine_kernel(part_hbm, mod_hbm, out_hbm, a_v, b_v, m_v):
    w = _worker_id()
    ubase = w * UPW
    pltpu.sync_copy(part_hbm.at[0].at[pl.ds(ubase, UPW)], a_v)
    pltpu.sync_copy(part_hbm.at[1].at[pl.ds(ubase, UPW)], b_v)
    pltpu.sync_copy(mod_hbm.at[pl.ds(ubase, UPW)], m_v)

    @pl.loop(0, UPW)
    def _row(r):
        for ci in range(NCH - 1):
            sl = pl.ds(ci * L, L)
            m_v[r, sl] = (a_v[r, sl] + b_v[r, sl]) * m_v[r, sl]

    pltpu.sync_copy(m_v, out_hbm.at[pl.ds(ubase, UPW)])


def _mod_body(ue_ref, le_ref, da_ref, wt_ref, out_ref):
    score = jax.nn.softmax(
        jnp.dot(ue_ref[...], le_ref[...].T, preferred_element_type=jnp.float32),
        axis=1)
    dw = jnp.dot(jax.nn.softmax(da_ref[...], axis=-1), wt_ref[...],
                 preferred_element_type=jnp.float32)
    out_ref[...] = 1.0 + jnp.dot(score, dw, preferred_element_type=jnp.float32)


_PB = 1000


def _pad_body(a_ref, b_ref, ap_ref, bp_ref):
    z = jnp.zeros((_PB, Dp - D), jnp.float32)
    ap_ref[...] = jnp.concatenate([a_ref[...], z], axis=1)
    bp_ref[...] = jnp.concatenate([b_ref[...], z], axis=1)


def _finalize_body(agg_ref, all_ref, rel_ref, nodep_ref, nodeout_ref):
    i = pl.program_id(0)
    news_f = jnp.where(i < N_NEWS // _PB, 1.0, 0.0)
    f = agg_ref[...] * (1.0 / N_NEIGH)
    f100 = f[:, :D] + all_ref[...] + rel_ref[0][None, :] * news_f
    nodep_ref[...] = jnp.concatenate(
        [f100, jnp.zeros((_PB, Dp - D), jnp.float32)], axis=1)
    nodeout_ref[...] = f100


def kernel(user_emb, all_embedding, entity_emb, relation_emb, latent_emb, weight,
           disen_weight_att, interact_vals, news_entities, news_relations,
           neigh_entities, neigh_relations, interact_rows, interact_cols):
    nidx = news_entities.reshape(NBLK_NEWS, BN, N_NEIGH).transpose(0, 2, 1).reshape(-1)
    ne_b = neigh_entities.reshape(NBLK_ENT, BN, N_NEIGH).transpose(0, 2, 1)
    nr_b = neigh_relations.reshape(NBLK_ENT, BN, N_NEIGH).transpose(0, 2, 1)
    ecidx = jnp.concatenate([ne_b, nr_b], axis=1).reshape(-1)
    rel_p = jnp.pad(relation_emb, ((0, 0), (0, Dp - D)))

    wt_p = jnp.pad(weight, ((0, 0), (0, Dp - D)))

    mod = pl.pallas_call(
        _mod_body,
        out_shape=jax.ShapeDtypeStruct((N_USERS, Dp), jnp.float32),
    )(user_emb, latent_emb, disen_weight_att, wt_p)

    ent_p, all_p = pl.pallas_call(
        _pad_body,
        grid=(N_ENTITY // _PB,),
        in_specs=[pl.BlockSpec((_PB, D), lambda i: (i, 0)),
                  pl.BlockSpec((_PB, D), lambda i: (i, 0))],
        out_specs=[pl.BlockSpec((_PB, Dp), lambda i: (i, 0)),
                   pl.BlockSpec((_PB, Dp), lambda i: (i, 0))],
        out_shape=[jax.ShapeDtypeStruct((N_ENTITY, Dp), jnp.float32),
                   jax.ShapeDtypeStruct((N_ENTITY, Dp), jnp.float32)],
    )(entity_emb, all_embedding)

    nnews_blk = N_NEWS // _PB
    agg = _node_kernel(ent_p, all_p, rel_p, nidx, ecidx)

    node_p, node_out = pl.pallas_call(
        _finalize_body,
        grid=(N_NODES // _PB,),
        in_specs=[
            pl.BlockSpec((_PB, Dp), lambda i: (i, 0)),
            pl.BlockSpec((_PB, D),
                         lambda i: (jnp.where(i < nnews_blk, i, i - nnews_blk), 0)),
            pl.BlockSpec((N_REL, D), lambda i: (0, 0)),
        ],
        out_specs=[pl.BlockSpec((_PB, Dp), lambda i: (i, 0)),
                   pl.BlockSpec((_PB, D), lambda i: (i, 0))],
        out_shape=[jax.ShapeDtypeStruct((N_NODES, Dp), jnp.float32),
                   jax.ShapeDtypeStruct((N_NODES, D), jnp.float32)],
    )(agg, all_embedding, relation_emb)

    part = _user_kernel(node_p, interact_cols, interact_vals, interact_rows)
    user_p = _combine_kernel(part, mod)

    return (node_out, user_p[:, :D])
